# Initial kernel scaffold; baseline (speedup 1.0000x reference)
import jax
import jax.numpy as jnp
from jax.experimental import pallas as pl


def kernel(x, edge_index, W1, b1, W2, b2, Wout, bout):
    raise NotImplementedError("write your pallas kernel here")


# speedup vs baseline: 13.1830x; 13.1830x over previous
import functools

import jax
import jax.numpy as jnp
from jax import lax
from jax.experimental import pallas as pl
from jax.experimental.pallas import tpu as pltpu
from jax.experimental.pallas import tpu_sc as plsc

N = 10000
E = 320000
D = 128
NC, NS, L = 2, 16, 16
NW = NC * NS
CH = 128
NCH = 79
EPW = NCH * CH
EP = NW * EPW
NPAD = 10240
PT = NPAD // NS
ZR = 64

_mesh = plsc.VectorSubcoreMesh(core_axis_name="c", subcore_axis_name="s")


@functools.partial(
    pl.kernel,
    out_type=jax.ShapeDtypeStruct((NC, NPAD), jnp.float32),
    mesh=_mesh,
    scratch_types=[
        pltpu.VMEM((NCH, CH), jnp.int32),
        pltpu.VMEM((CH,), jnp.float32),
        pltpu.VMEM((PT,), jnp.float32),
        pltpu.VMEM_SHARED((NPAD,), jnp.float32),
    ],
)
def _deg_kernel(dst_hbm, out_hbm, idx_v, ones_v, zero_v, acc_sh):
    cid = lax.axis_index("c")
    sid = lax.axis_index("s")
    wid = cid * NS + sid

    def fill_ones(i, c):
        ones_v[pl.ds(i * L, L)] = jnp.full((L,), 1.0, jnp.float32)
        return c

    lax.fori_loop(0, CH // L, fill_ones, 0)

    def fill_zero(i, c):
        zero_v[pl.ds(i * L, L)] = jnp.zeros((L,), jnp.float32)
        return c

    lax.fori_loop(0, PT // L, fill_zero, 0)

    base = sid * PT
    pltpu.sync_copy(zero_v, acc_sh.at[pl.ds(base, PT)])
    plsc.subcore_barrier()

    pltpu.sync_copy(dst_hbm.at[wid], idx_v)

    def body(c, carry):
        pltpu.sync_copy(ones_v, acc_sh.at[idx_v.at[c]], add=True)
        return carry

    lax.fori_loop(0, NCH, body, 0)
    plsc.subcore_barrier()
    pltpu.sync_copy(acc_sh.at[pl.ds(base, PT)], out_hbm.at[cid, pl.ds(base, PT)])


@functools.partial(
    pl.kernel,
    out_type=jax.ShapeDtypeStruct((NC, NPAD, D), jnp.float32),
    mesh=_mesh,
    scratch_types=[
        pltpu.VMEM((NCH, CH), jnp.int32),
        pltpu.VMEM((NCH, CH), jnp.int32),
        pltpu.VMEM((CH, D), jnp.float32),
        pltpu.VMEM((ZR, D), jnp.float32),
        pltpu.VMEM_SHARED((NPAD, D), jnp.float32),
        pltpu.SemaphoreType.DMA,
    ],
)
def _seg_kernel(ht_hbm, src_hbm, dst_hbm, out_hbm,
                src_v, dst_v, rows_v, zero_v, acc_sh, sem):
    cid = lax.axis_index("c")
    sid = lax.axis_index("s")
    wid = cid * NS + sid

    def fill_zero(i, c):
        for j in range(D // L):
            zero_v[i, pl.ds(j * L, L)] = jnp.zeros((L,), jnp.float32)
        return c

    lax.fori_loop(0, ZR, fill_zero, 0)

    base = sid * PT
    for k in range(PT // ZR):
        pltpu.sync_copy(zero_v, acc_sh.at[pl.ds(base + k * ZR, ZR)])
    plsc.subcore_barrier()

    pltpu.sync_copy(src_hbm.at[wid], src_v)
    pltpu.sync_copy(dst_hbm.at[wid], dst_v)

    def body(c, carry):
        pltpu.async_copy(ht_hbm.at[src_v.at[c]], rows_v, sem).wait()
        pltpu.sync_copy(rows_v, acc_sh.at[dst_v.at[c]], add=True)
        return carry

    lax.fori_loop(0, NCH, body, 0)
    plsc.subcore_barrier()
    pltpu.sync_copy(acc_sh.at[pl.ds(base, PT)],
                    out_hbm.at[cid, pl.ds(base, PT)])


def _mm_scale_body(x_ref, w_ref, deg_ref, ht_ref, dinv_ref):
    deg = deg_ref[0] + deg_ref[1]
    dinv = lax.rsqrt(deg[:N] + 1.0)
    h = jnp.dot(x_ref[...], w_ref[...], preferred_element_type=jnp.float32)
    ht_ref[...] = h * dinv
    dinv_ref[...] = dinv


def _layer_body(acc_ref, ht_ref, dinv_ref, b_ref, w_ref, out_ref):
    s = acc_ref[0, :N, :] + acc_ref[1, :N, :] + ht_ref[...]
    dinv = dinv_ref[...]
    h = jnp.maximum(s * dinv + b_ref[...], 0.0)
    out_ref[...] = jnp.dot(h, w_ref[...],
                           preferred_element_type=jnp.float32) * dinv


def _final_body(acc_ref, ht_ref, dinv_ref, b_ref, w_ref, bout_ref, out_ref):
    s = acc_ref[0, :N, :] + acc_ref[1, :N, :] + ht_ref[...]
    h = jnp.maximum(s * dinv_ref[...] + b_ref[...], 0.0)
    out_ref[...] = jnp.dot(h, w_ref[...],
                           preferred_element_type=jnp.float32) + bout_ref[...]


def kernel(x, edge_index, W1, b1, W2, b2, Wout, bout):
    ei = edge_index.astype(jnp.int32)
    pad = EP - E
    srcp = jnp.concatenate([ei[0], jnp.zeros((pad,), jnp.int32)])
    dstp = jnp.concatenate([ei[1], jnp.full((pad,), N, jnp.int32)])
    srcp = srcp.reshape(NW, NCH, CH)
    dstp = dstp.reshape(NW, NCH, CH)

    degraw = _deg_kernel(dstp).reshape(NC, NPAD, 1)

    h1t, dinv = pl.pallas_call(
        _mm_scale_body,
        out_shape=(jax.ShapeDtypeStruct((N, D), jnp.float32),
                   jax.ShapeDtypeStruct((N, 1), jnp.float32)),
    )(x, W1, degraw)

    acc1 = _seg_kernel(h1t, srcp, dstp)

    h2t = pl.pallas_call(
        _layer_body,
        out_shape=jax.ShapeDtypeStruct((N, D), jnp.float32),
    )(acc1, h1t, dinv, b1.reshape(1, D), W2)

    acc2 = _seg_kernel(h2t, srcp, dstp)

    out = pl.pallas_call(
        _final_body,
        out_shape=jax.ShapeDtypeStruct((N, 1), jnp.float32),
    )(acc2, h2t, dinv, b2.reshape(1, D), Wout, bout.reshape(1, 1))
    return out
